# Initial kernel scaffold; baseline (speedup 1.0000x reference)
import jax
import jax.numpy as jnp
from jax.experimental import pallas as pl


def kernel(x, vertex_id, W1, b1, We, Wv):
    raise NotImplementedError("write your pallas kernel here")


# speedup vs baseline: 1.0185x; 1.0185x over previous
import functools
import jax
import jax.numpy as jnp
from jax import lax
from jax.experimental import pallas as pl
from jax.experimental.pallas import tpu as pltpu


def _mm2_body(x_ref, w1t_ref, b1_ref, wet_ref, z_ref, ze_ref):
    x = x_ref[...]
    z_ref[...] = (
        jnp.dot(x, w1t_ref[...], preferred_element_type=jnp.float32) + b1_ref[...]
    )
    ze_ref[...] = jnp.dot(x, wet_ref[...], preferred_element_type=jnp.float32)


def _dual_matmul(x, W1, b1, We, block_rows=2560):
    E, C = x.shape
    M = W1.shape[0]
    O = We.shape[0]
    grid = (E // block_rows,)
    return pl.pallas_call(
        _mm2_body,
        grid=grid,
        in_specs=[
            pl.BlockSpec((block_rows, C), lambda i: (i, 0)),
            pl.BlockSpec((C, M), lambda i: (0, 0)),
            pl.BlockSpec((1, M), lambda i: (0, 0)),
            pl.BlockSpec((C, O), lambda i: (0, 0)),
        ],
        out_specs=[
            pl.BlockSpec((block_rows, M), lambda i: (i, 0)),
            pl.BlockSpec((block_rows, O), lambda i: (i, 0)),
        ],
        out_shape=[
            jax.ShapeDtypeStruct((E, M), jnp.float32),
            jax.ShapeDtypeStruct((E, O), jnp.float32),
        ],
    )(x, W1.T, b1.reshape(1, -1), We.T)


def _mm_body(a_ref, bt_ref, o_ref):
    o_ref[...] = jnp.dot(a_ref[...], bt_ref[...], preferred_element_type=jnp.float32)


def _small_matmul(a, Wv, block_rows=2000):
    V, M = a.shape
    O = Wv.shape[0]
    return pl.pallas_call(
        _mm_body,
        grid=(V // block_rows,),
        in_specs=[
            pl.BlockSpec((block_rows, M), lambda i: (i, 0)),
            pl.BlockSpec((M, O), lambda i: (0, 0)),
        ],
        out_specs=pl.BlockSpec((block_rows, O), lambda i: (i, 0)),
        out_shape=jax.ShapeDtypeStruct((V, O), jnp.float32),
    )(a, Wv.T)


def kernel(x, vertex_id, W1, b1, We, Wv):
    E, C = x.shape
    V = 10000

    z, z_edge = _dual_matmul(x, W1, b1, We)

    z_vertex = jax.ops.segment_max(z, vertex_id, num_segments=V)
    z_vertex = jnp.where(jnp.isfinite(z_vertex), z_vertex, 0.0)

    zv2 = _small_matmul(z_vertex, Wv)

    return z_edge + jnp.take(zv2, vertex_id, axis=0)


# speedup vs baseline: 1.4249x; 1.3990x over previous
import functools
import jax
import jax.numpy as jnp
from jax import lax
from jax.experimental import pallas as pl
from jax.experimental.pallas import tpu as pltpu
from jax.experimental.pallas import tpu_sc as plsc

NW = 32
VPW = 313
VPAD = NW * VPW
T = 80
NEG = float("-inf")


def _sload(ref, i):
    return ref[pl.ds(i, 16)][0]


def _mm2_body(x_ref, w1t_ref, b1_ref, wet_ref, z_ref, ze_ref):
    x = x_ref[...]
    z_ref[...] = (
        jnp.dot(x, w1t_ref[...], preferred_element_type=jnp.float32) + b1_ref[...]
    )
    ze_ref[...] = jnp.dot(x, wet_ref[...], preferred_element_type=jnp.float32)


def _dual_matmul(x, W1, b1, We, block_rows=2560):
    E, C = x.shape
    M = W1.shape[0]
    O = We.shape[0]
    return pl.pallas_call(
        _mm2_body,
        grid=(E // block_rows,),
        in_specs=[
            pl.BlockSpec((block_rows, C), lambda i: (i, 0)),
            pl.BlockSpec((C, M), lambda i: (0, 0)),
            pl.BlockSpec((1, M), lambda i: (0, 0)),
            pl.BlockSpec((C, O), lambda i: (0, 0)),
        ],
        out_specs=[
            pl.BlockSpec((block_rows, M), lambda i: (i, 0)),
            pl.BlockSpec((block_rows, O), lambda i: (i, 0)),
        ],
        out_shape=[
            jax.ShapeDtypeStruct((E, M), jnp.float32),
            jax.ShapeDtypeStruct((E, O), jnp.float32),
        ],
    )(x, W1.T, b1.reshape(1, -1), We.T)


def _mm_body(a_ref, bt_ref, o_ref):
    o_ref[...] = jnp.dot(a_ref[...], bt_ref[...], preferred_element_type=jnp.float32)


def _small_matmul(a, Wv, block_rows=2504):
    V, M = a.shape
    O = Wv.shape[0]
    return pl.pallas_call(
        _mm_body,
        grid=(V // block_rows,),
        in_specs=[
            pl.BlockSpec((block_rows, M), lambda i: (i, 0)),
            pl.BlockSpec((M, O), lambda i: (0, 0)),
        ],
        out_specs=pl.BlockSpec((block_rows, O), lambda i: (i, 0)),
        out_shape=jax.ShapeDtypeStruct((V, O), jnp.float32),
    )(a, Wv.T)


def _seg_max_sc(z_flat, E, vid_pad, bnd):
    mesh = plsc.VectorSubcoreMesh(core_axis_name="c", subcore_axis_name="s")

    @functools.partial(
        pl.kernel,
        out_type=jax.ShapeDtypeStruct((VPAD * 128,), jnp.float32),
        mesh=mesh,
        scratch_types=[
            pltpu.VMEM((48,), jnp.int32),
            pltpu.VMEM((104,), jnp.int32),
            pltpu.VMEM((T * 128,), jnp.float32),
            pltpu.VMEM((VPW * 128,), jnp.float32),
        ],
    )
    def seg_kernel(z_hbm, vidp_hbm, bnd_hbm, zvert_hbm, bnd_v, idx_v, rows_v, loc_v):
        w = lax.axis_index("s") * 2 + lax.axis_index("c")
        v0 = w * VPW
        pltpu.sync_copy(bnd_hbm, bnd_v.at[pl.ds(0, 40)])
        e0 = _sload(bnd_v, w)
        e1 = _sload(bnd_v, w + 1)
        cnt = e1 - e0

        def init_body(i, c):
            loc_v[pl.ds(i * 16, 16)] = jnp.full((16,), NEG, jnp.float32)
            return c
        lax.fori_loop(0, VPW * 8, init_body, 0)

        neg_row = tuple(jnp.full((16,), NEG, jnp.float32) for _ in range(8))

        def close_seg(prev_vid, acc):
            @pl.when(prev_vid >= 0)
            def _():
                po = (prev_vid - v0) * 128
                for g in range(8):
                    loc_v[pl.ds(po + g * 16, 16)] = acc[g]

        def edge_body(j, carry):
            prev_vid, off = carry[0], carry[1]
            acc = carry[2:]
            vj = _sload(idx_v, off + j)
            changed = vj != prev_vid

            @pl.when(changed)
            def _():
                close_seg(prev_vid, acc)

            new_acc = []
            for g in range(8):
                row = rows_v[pl.ds(j * 128 + g * 16, 16)]
                new_acc.append(jnp.where(changed, row, jnp.maximum(acc[g], row)))
            return (jnp.where(changed, vj, prev_vid), off) + tuple(new_acc)

        def do_tile(s_t, j_lo, j_hi, carry):
            s8 = (s_t // 8) * 8
            off = s_t - s8
            pltpu.sync_copy(vidp_hbm.at[pl.ds(s8, 88)], idx_v.at[pl.ds(0, 88)])
            pltpu.sync_copy(z_hbm.at[pl.ds(s_t * 128, T * 128)], rows_v)
            carry = (carry[0], off) + carry[2:]
            carry = lax.fori_loop(j_lo, j_hi, edge_body, carry)
            return carry

        def tile_body(t, carry):
            return do_tile(e0 + t * T, 0, T, carry)

        nfull = cnt // T
        carry = (jnp.int32(-1), jnp.int32(0)) + neg_row
        carry = lax.fori_loop(0, nfull, tile_body, carry)

        rem = cnt - nfull * T

        @pl.when(rem > 0)
        def _():
            des = e0 + nfull * T
            s_t = jnp.minimum(des, E - T)
            skip = des - s_t
            c2 = do_tile(s_t, skip, skip + rem, carry)
            close_seg(c2[0], c2[2:])

        @pl.when(rem == 0)
        def _():
            close_seg(carry[0], carry[2:])

        pltpu.sync_copy(loc_v, zvert_hbm.at[pl.ds(v0 * 128, VPW * 128)])

    return seg_kernel(z_flat, vid_pad, bnd)


def _gather_add_sc(ze_flat, E, zv2_flat, vid_pad, bnd):
    mesh = plsc.VectorSubcoreMesh(core_axis_name="c", subcore_axis_name="s")

    @functools.partial(
        pl.kernel,
        out_type=jax.ShapeDtypeStruct((E * 128,), jnp.float32),
        mesh=mesh,
        scratch_types=[
            pltpu.VMEM((48,), jnp.int32),
            pltpu.VMEM((104,), jnp.int32),
            pltpu.VMEM((T * 128,), jnp.float32),
            pltpu.VMEM((VPW * 128,), jnp.float32),
        ],
    )
    def ga_kernel(ze_hbm, zv2_hbm, vidp_hbm, bnd_hbm, out_hbm, bnd_v, idx_v, rows_v, loc_v):
        w = lax.axis_index("s") * 2 + lax.axis_index("c")
        v0 = w * VPW
        pltpu.sync_copy(bnd_hbm, bnd_v.at[pl.ds(0, 40)])
        e0 = _sload(bnd_v, w)
        e1 = _sload(bnd_v, w + 1)
        cnt = e1 - e0

        pltpu.sync_copy(zv2_hbm.at[pl.ds(v0 * 128, VPW * 128)], loc_v)

        def proc_edges(off, j_lo, j_hi):
            def edge_body(j, c):
                vj = _sload(idx_v, off + j)
                po = (vj - v0) * 128
                ro = j * 128
                for g in range(8):
                    rows_v[pl.ds(ro + g * 16, 16)] = (
                        rows_v[pl.ds(ro + g * 16, 16)] + loc_v[pl.ds(po + g * 16, 16)]
                    )
                return c
            lax.fori_loop(j_lo, j_hi, edge_body, 0)

        def load_tile(s_t):
            s8 = (s_t // 8) * 8
            pltpu.sync_copy(vidp_hbm.at[pl.ds(s8, 88)], idx_v.at[pl.ds(0, 88)])
            pltpu.sync_copy(ze_hbm.at[pl.ds(s_t * 128, T * 128)], rows_v)
            return s_t - s8

        def tile_body(t, c):
            s_t = e0 + t * T
            off = load_tile(s_t)
            proc_edges(off, 0, T)
            pltpu.sync_copy(rows_v, out_hbm.at[pl.ds(s_t * 128, T * 128)])
            return c

        nfull = cnt // T
        lax.fori_loop(0, nfull, tile_body, 0)
        rem = cnt - nfull * T

        @pl.when(rem > 0)
        def _():
            @pl.when(cnt >= T)
            def _():
                s_t = e1 - T
                off = load_tile(s_t)
                proc_edges(off, 0, T)
                pltpu.sync_copy(rows_v, out_hbm.at[pl.ds(s_t * 128, T * 128)])

            @pl.when(cnt < T)
            def _():
                s_t = jnp.minimum(e0, E - T)
                skip = e0 - s_t
                off = load_tile(s_t)
                proc_edges(off, skip, skip + rem)

                def row_out(j, c):
                    pltpu.sync_copy(
                        rows_v.at[pl.ds(j * 128, 128)],
                        out_hbm.at[pl.ds((s_t + j) * 128, 128)],
                    )
                    return c
                lax.fori_loop(skip, skip + rem, row_out, 0)

    return ga_kernel(ze_flat, zv2_flat, vid_pad, bnd)


def kernel(x, vertex_id, W1, b1, We, Wv):
    E, C = x.shape
    vid = vertex_id.astype(jnp.int32)

    bnd = jnp.searchsorted(
        vid, (jnp.arange(33, dtype=jnp.int32) * VPW).astype(jnp.int32), side="left"
    ).astype(jnp.int32)
    bnd = jnp.concatenate([bnd, jnp.full((7,), E, jnp.int32)])
    vid_pad = jnp.concatenate([vid, jnp.zeros((88,), jnp.int32)])

    z, z_edge = _dual_matmul(x, W1, b1, We)
    zvert_flat = _seg_max_sc(z.reshape(-1), E, vid_pad, bnd)
    zvert = zvert_flat.reshape(VPAD, 128)
    zv2 = _small_matmul(zvert, Wv)
    out_flat = _gather_add_sc(z_edge.reshape(-1), E, zv2.reshape(-1), vid_pad, bnd)
    return out_flat.reshape(E, 128)


# speedup vs baseline: 2.0493x; 1.4382x over previous
import functools
import jax
import jax.numpy as jnp
from jax import lax
from jax.experimental import pallas as pl
from jax.experimental.pallas import tpu as pltpu
from jax.experimental.pallas import tpu_sc as plsc

NW = 32
VPW = 313
VPAD = NW * VPW
T = 128
NEG = float("-inf")


def _sload(ref, i):
    return ref[pl.ds(i, 16)][0]


def _mm2_body(x_ref, w1t_ref, b1_ref, wet_ref, z_ref, ze_ref):
    x = x_ref[...]
    z_ref[...] = (
        jnp.dot(x, w1t_ref[...], preferred_element_type=jnp.float32) + b1_ref[...]
    )
    ze_ref[...] = jnp.dot(x, wet_ref[...], preferred_element_type=jnp.float32)


def _dual_matmul(x, W1, b1, We, block_rows=2560):
    E, C = x.shape
    M = W1.shape[0]
    O = We.shape[0]
    return pl.pallas_call(
        _mm2_body,
        grid=(E // block_rows,),
        in_specs=[
            pl.BlockSpec((block_rows, C), lambda i: (i, 0)),
            pl.BlockSpec((C, M), lambda i: (0, 0)),
            pl.BlockSpec((1, M), lambda i: (0, 0)),
            pl.BlockSpec((C, O), lambda i: (0, 0)),
        ],
        out_specs=[
            pl.BlockSpec((block_rows, M), lambda i: (i, 0)),
            pl.BlockSpec((block_rows, O), lambda i: (i, 0)),
        ],
        out_shape=[
            jax.ShapeDtypeStruct((E, M), jnp.float32),
            jax.ShapeDtypeStruct((E, O), jnp.float32),
        ],
    )(x, W1.T, b1.reshape(1, -1), We.T)


def _mm_body(a_ref, bt_ref, o_ref):
    o_ref[...] = jnp.dot(a_ref[...], bt_ref[...], preferred_element_type=jnp.float32)


def _small_matmul(a, Wv, block_rows=2504):
    V, M = a.shape
    O = Wv.shape[0]
    return pl.pallas_call(
        _mm_body,
        grid=(V // block_rows,),
        in_specs=[
            pl.BlockSpec((block_rows, M), lambda i: (i, 0)),
            pl.BlockSpec((M, O), lambda i: (0, 0)),
        ],
        out_specs=pl.BlockSpec((block_rows, O), lambda i: (i, 0)),
        out_shape=jax.ShapeDtypeStruct((V, O), jnp.float32),
    )(a, Wv.T)


TI = T + 8
IDS_STRIDE = 160


def _seg_max_sc(z_flat, E, vid_pad, bnd):
    mesh = plsc.VectorSubcoreMesh(core_axis_name="c", subcore_axis_name="s")

    @functools.partial(
        pl.kernel,
        out_type=jax.ShapeDtypeStruct((VPAD * 128,), jnp.float32),
        mesh=mesh,
        scratch_types=[
            pltpu.VMEM((48,), jnp.int32),
            pltpu.VMEM((2 * IDS_STRIDE,), jnp.int32),
            pltpu.VMEM((2 * T * 128,), jnp.float32),
            pltpu.VMEM((VPW * 128,), jnp.float32),
            pltpu.SemaphoreType.DMA,
            pltpu.SemaphoreType.DMA,
        ],
    )
    def seg_kernel(z_hbm, vidp_hbm, bnd_hbm, zvert_hbm,
                   bnd_v, ids_v, rows_v, loc_v, si, sr):
        w = lax.axis_index("s") * 2 + lax.axis_index("c")
        v0 = w * VPW
        pltpu.sync_copy(bnd_hbm, bnd_v.at[pl.ds(0, 40)])
        e0 = _sload(bnd_v, w)
        e1 = _sload(bnd_v, w + 1)
        cnt = e1 - e0
        ntot = (cnt + T - 1) // T

        def init_body(i, c):
            loc_v[pl.ds(i * 16, 16)] = jnp.full((16,), NEG, jnp.float32)
            return c
        lax.fori_loop(0, VPW * 8, init_body, 0)

        def s_of(t):
            return jnp.minimum(e0 + t * T, E - T)

        def start_in(t, slot):
            s = s_of(t)
            s8 = (s // 8) * 8
            pltpu.async_copy(
                vidp_hbm.at[pl.ds(s8, TI)],
                ids_v.at[pl.ds(slot * IDS_STRIDE, TI)], si)
            pltpu.async_copy(
                z_hbm.at[pl.ds(s * 128, T * 128)],
                rows_v.at[pl.ds(slot * T * 128, T * 128)], sr)

        def wait_in():
            pltpu.make_async_copy(
                vidp_hbm.at[pl.ds(0, TI)], ids_v.at[pl.ds(0, TI)], si).wait()
            pltpu.make_async_copy(
                z_hbm.at[pl.ds(0, T * 128)], rows_v.at[pl.ds(0, T * 128)], sr).wait()

        def close_seg(prev_vid, acc):
            @pl.when(prev_vid >= 0)
            def _():
                po = (prev_vid - v0) * 128
                for g in range(8):
                    loc_v[pl.ds(po + g * 16, 16)] = acc[g]

        def process(t, slot, carry):
            s = s_of(t)
            off = slot * IDS_STRIDE + s - (s // 8) * 8
            base_r = slot * (T * 128)
            skip = (e0 + t * T) - s
            j_hi = skip + jnp.minimum(T, cnt - t * T)

            def edge_body(j, c):
                prev_vid = c[0]
                acc = c[1:]
                vj = _sload(ids_v, off + j)
                changed = vj != prev_vid

                @pl.when(changed)
                def _():
                    close_seg(prev_vid, acc)

                new_acc = []
                for g in range(8):
                    row = rows_v[pl.ds(base_r + j * 128 + g * 16, 16)]
                    new_acc.append(jnp.where(changed, row, jnp.maximum(acc[g], row)))
                return (jnp.where(changed, vj, prev_vid),) + tuple(new_acc)

            return lax.fori_loop(skip, j_hi, edge_body, carry)

        @pl.when(ntot > 0)
        def _():
            start_in(0, 0)

        neg_row = tuple(jnp.full((16,), NEG, jnp.float32) for _ in range(8))
        carry = (jnp.int32(-1),) + neg_row

        def loop_body(t, c):
            wait_in()

            @pl.when(t + 1 < ntot)
            def _():
                start_in(t + 1, (t + 1) % 2)

            return process(t, t % 2, c)

        carry = lax.fori_loop(0, ntot, loop_body, carry)
        close_seg(carry[0], carry[1:])

        pltpu.sync_copy(loc_v, zvert_hbm.at[pl.ds(v0 * 128, VPW * 128)])

    return seg_kernel(z_flat, vid_pad, bnd)


def _gather_add_sc(ze_flat, E, zv2_flat, vid_pad, bnd):
    mesh = plsc.VectorSubcoreMesh(core_axis_name="c", subcore_axis_name="s")

    @functools.partial(
        pl.kernel,
        out_type=jax.ShapeDtypeStruct((E * 128,), jnp.float32),
        mesh=mesh,
        scratch_types=[
            pltpu.VMEM((48,), jnp.int32),
            pltpu.VMEM((TI + 16,), jnp.int32),
            pltpu.VMEM((TI + 16,), jnp.int32),
            pltpu.VMEM((T * 128,), jnp.float32),
            pltpu.VMEM((T * 128,), jnp.float32),
            pltpu.VMEM((T * 128,), jnp.float32),
            pltpu.VMEM((T * 128,), jnp.float32),
            pltpu.VMEM((VPW * 128,), jnp.float32),
            pltpu.SemaphoreType.DMA,
            pltpu.SemaphoreType.DMA,
            pltpu.SemaphoreType.DMA,
            pltpu.SemaphoreType.DMA,
            pltpu.SemaphoreType.DMA,
            pltpu.SemaphoreType.DMA,
        ],
    )
    def ga_kernel(ze_hbm, zv2_hbm, vidp_hbm, bnd_hbm, out_hbm,
                  bnd_v, ids0, ids1, rows0, rows1, ob0, ob1, loc_v,
                  si0, si1, sr0, sr1, so0, so1):
        ids_b = (ids0, ids1)
        rows_b = (rows0, rows1)
        out_b = (ob0, ob1)
        sem_i = (si0, si1)
        sem_r = (sr0, sr1)
        sem_o = (so0, so1)

        w = lax.axis_index("s") * 2 + lax.axis_index("c")
        v0 = w * VPW
        pltpu.sync_copy(bnd_hbm, bnd_v.at[pl.ds(0, 40)])
        e0 = _sload(bnd_v, w)
        e1 = _sload(bnd_v, w + 1)
        cnt = e1 - e0
        ntot = jnp.where(cnt >= T, (cnt + T - 1) // T, 0)

        pltpu.sync_copy(zv2_hbm.at[pl.ds(v0 * 128, VPW * 128)], loc_v)

        def s_of(t):
            return jnp.minimum(e0 + t * T, e1 - T)

        def start_in(t, b):
            s = s_of(t)
            s8 = (s // 8) * 8
            pltpu.async_copy(vidp_hbm.at[pl.ds(s8, TI)], ids_b[b].at[pl.ds(0, TI)], sem_i[b])
            pltpu.async_copy(ze_hbm.at[pl.ds(s * 128, T * 128)], rows_b[b], sem_r[b])

        def wait_in(b):
            pltpu.make_async_copy(vidp_hbm.at[pl.ds(0, TI)], ids_b[b].at[pl.ds(0, TI)], sem_i[b]).wait()
            pltpu.make_async_copy(ze_hbm.at[pl.ds(0, T * 128)], rows_b[b], sem_r[b]).wait()

        def wait_out(b):
            pltpu.make_async_copy(out_b[b], out_hbm.at[pl.ds(0, T * 128)], sem_o[b]).wait()

        def proc_edges(ids_r, rows_r, dst_r, off, j_lo, j_hi):
            def edge_body(j, c):
                vj = _sload(ids_r, off + j)
                po = (vj - v0) * 128
                ro = j * 128
                for g in range(8):
                    dst_r[pl.ds(ro + g * 16, 16)] = (
                        rows_r[pl.ds(ro + g * 16, 16)] + loc_v[pl.ds(po + g * 16, 16)]
                    )
                return c
            lax.fori_loop(j_lo, j_hi, edge_body, 0)

        def turn(t, b):
            wait_in(b)

            @pl.when(t + 1 < ntot)
            def _():
                start_in(t + 1, 1 - b)

            @pl.when(t >= 2)
            def _():
                wait_out(b)

            s = s_of(t)
            off = s - (s // 8) * 8
            proc_edges(ids_b[b], rows_b[b], out_b[b], off, 0, T)
            pltpu.async_copy(out_b[b], out_hbm.at[pl.ds(s * 128, T * 128)], sem_o[b])

        @pl.when(ntot > 0)
        def _():
            start_in(0, 0)

        def loop_body(t, c):
            lax.cond(
                t % 2 == 0,
                lambda: turn(t, 0),
                lambda: turn(t, 1),
            )
            return c

        lax.fori_loop(0, ntot, loop_body, 0)

        @pl.when(ntot >= 1)
        def _():
            wait_out(0)

        @pl.when(ntot >= 2)
        def _():
            wait_out(1)

        @pl.when((cnt > 0) & (cnt < T))
        def _():
            s = jnp.minimum(e0, E - T)
            s8 = (s // 8) * 8
            off = s - s8
            skip = e0 - s
            pltpu.sync_copy(vidp_hbm.at[pl.ds(s8, TI)], ids0.at[pl.ds(0, TI)])
            pltpu.sync_copy(ze_hbm.at[pl.ds(s * 128, T * 128)], rows0)
            proc_edges(ids0, rows0, ob0, off, skip, skip + cnt)

            def row_out(j, c):
                pltpu.sync_copy(
                    ob0.at[pl.ds(j * 128, 128)],
                    out_hbm.at[pl.ds((s + j) * 128, 128)],
                )
                return c
            lax.fori_loop(skip, skip + cnt, row_out, 0)

    return ga_kernel(ze_flat, zv2_flat, vid_pad, bnd)


def kernel(x, vertex_id, W1, b1, We, Wv):
    E, C = x.shape
    vid = vertex_id.astype(jnp.int32)

    bnd = jnp.searchsorted(
        vid, (jnp.arange(33, dtype=jnp.int32) * VPW).astype(jnp.int32), side="left"
    ).astype(jnp.int32)
    bnd = jnp.concatenate([bnd, jnp.full((7,), E, jnp.int32)])
    vid_pad = jnp.concatenate([vid, jnp.zeros((88,), jnp.int32)])

    z, z_edge = _dual_matmul(x, W1, b1, We)
    zvert_flat = _seg_max_sc(z.reshape(-1), E, vid_pad, bnd)
    zvert = zvert_flat.reshape(VPAD, 128)
    zv2 = _small_matmul(zvert, Wv)
    out_flat = _gather_add_sc(z_edge.reshape(-1), E, zv2.reshape(-1), vid_pad, bnd)
    return out_flat.reshape(E, 128)


# speedup vs baseline: 2.1732x; 1.0605x over previous
import functools
import jax
import jax.numpy as jnp
from jax import lax
from jax.experimental import pallas as pl
from jax.experimental.pallas import tpu as pltpu
from jax.experimental.pallas import tpu_sc as plsc

NW = 32
VPW = 313
VPAD = NW * VPW
T = 128
NEG = float("-inf")


def _sload(ref, i):
    return ref[pl.ds(i, 16)][0]


def _mm2_body(x_ref, w1t_ref, b1_ref, wet_ref, z_ref, ze_ref):
    x = x_ref[...]
    z_ref[...] = (
        jnp.dot(x, w1t_ref[...], preferred_element_type=jnp.float32) + b1_ref[...]
    )
    ze_ref[...] = jnp.dot(x, wet_ref[...], preferred_element_type=jnp.float32)


def _dual_matmul(x, W1, b1, We, block_rows=2560):
    E, C = x.shape
    M = W1.shape[0]
    O = We.shape[0]
    return pl.pallas_call(
        _mm2_body,
        grid=(E // block_rows,),
        in_specs=[
            pl.BlockSpec((block_rows, C), lambda i: (i, 0)),
            pl.BlockSpec((C, M), lambda i: (0, 0)),
            pl.BlockSpec((1, M), lambda i: (0, 0)),
            pl.BlockSpec((C, O), lambda i: (0, 0)),
        ],
        out_specs=[
            pl.BlockSpec((block_rows, M), lambda i: (i, 0)),
            pl.BlockSpec((block_rows, O), lambda i: (i, 0)),
        ],
        out_shape=[
            jax.ShapeDtypeStruct((E, M), jnp.float32),
            jax.ShapeDtypeStruct((E, O), jnp.float32),
        ],
    )(x, W1.T, b1.reshape(1, -1), We.T)


def _mm_body(a_ref, bt_ref, o_ref):
    o_ref[...] = jnp.dot(a_ref[...], bt_ref[...], preferred_element_type=jnp.float32)


def _small_matmul(a, Wv, block_rows=2504):
    V, M = a.shape
    O = Wv.shape[0]
    return pl.pallas_call(
        _mm_body,
        grid=(V // block_rows,),
        in_specs=[
            pl.BlockSpec((block_rows, M), lambda i: (i, 0)),
            pl.BlockSpec((M, O), lambda i: (0, 0)),
        ],
        out_specs=pl.BlockSpec((block_rows, O), lambda i: (i, 0)),
        out_shape=jax.ShapeDtypeStruct((V, O), jnp.float32),
    )(a, Wv.T)


TI = T + 8
IDS_STRIDE = 160


def _seg_max_sc(z_flat, E, vid_pad, bnd):
    mesh = plsc.VectorSubcoreMesh(core_axis_name="c", subcore_axis_name="s")

    @functools.partial(
        pl.kernel,
        out_type=jax.ShapeDtypeStruct((VPAD * 128,), jnp.float32),
        mesh=mesh,
        scratch_types=[
            pltpu.VMEM((48,), jnp.int32),
            pltpu.VMEM((2 * IDS_STRIDE,), jnp.int32),
            pltpu.VMEM((2 * T * 128,), jnp.float32),
            pltpu.VMEM((VPW * 128,), jnp.float32),
            pltpu.SemaphoreType.DMA,
            pltpu.SemaphoreType.DMA,
        ],
    )
    def seg_kernel(z_hbm, vidp_hbm, bnd_hbm, zvert_hbm,
                   bnd_v, ids_v, rows_v, loc_v, si, sr):
        w = lax.axis_index("s") * 2 + lax.axis_index("c")
        v0 = w * VPW
        pltpu.sync_copy(bnd_hbm, bnd_v.at[pl.ds(0, 40)])
        e0 = _sload(bnd_v, w)
        e1 = _sload(bnd_v, w + 1)
        cnt = e1 - e0
        ntot = (cnt + T - 1) // T

        def init_body(i, c):
            loc_v[pl.ds(i * 16, 16)] = jnp.full((16,), NEG, jnp.float32)
            return c
        lax.fori_loop(0, VPW * 8, init_body, 0)

        def s_of(t):
            return jnp.minimum(e0 + t * T, E - T)

        def start_in(t, slot):
            s = s_of(t)
            s8 = (s // 8) * 8
            pltpu.async_copy(
                vidp_hbm.at[pl.ds(s8, TI)],
                ids_v.at[pl.ds(slot * IDS_STRIDE, TI)], si)
            pltpu.async_copy(
                z_hbm.at[pl.ds(s * 128, T * 128)],
                rows_v.at[pl.ds(slot * T * 128, T * 128)], sr)

        def wait_in():
            pltpu.make_async_copy(
                vidp_hbm.at[pl.ds(0, TI)], ids_v.at[pl.ds(0, TI)], si).wait()
            pltpu.make_async_copy(
                z_hbm.at[pl.ds(0, T * 128)], rows_v.at[pl.ds(0, T * 128)], sr).wait()

        def close_seg(prev_vid, acc):
            @pl.when(prev_vid >= 0)
            def _():
                po = (prev_vid - v0) * 128
                for g in range(8):
                    loc_v[pl.ds(po + g * 16, 16)] = acc[g]

        def process(t, slot, carry):
            s = s_of(t)
            off = slot * IDS_STRIDE + s - (s // 8) * 8
            base_r = slot * (T * 128)
            skip = (e0 + t * T) - s
            j_hi = skip + jnp.minimum(T, cnt - t * T)

            def edge_body(j, c):
                prev_vid = c[0]
                acc = c[1:]
                vj = _sload(ids_v, off + j)
                changed = vj != prev_vid

                @pl.when(changed)
                def _():
                    close_seg(prev_vid, acc)

                new_acc = []
                for g in range(8):
                    row = rows_v[pl.ds(base_r + j * 128 + g * 16, 16)]
                    new_acc.append(jnp.where(changed, row, jnp.maximum(acc[g], row)))
                return (jnp.where(changed, vj, prev_vid),) + tuple(new_acc)

            return lax.fori_loop(skip, j_hi, edge_body, carry)

        @pl.when(ntot > 0)
        def _():
            start_in(0, 0)

        neg_row = tuple(jnp.full((16,), NEG, jnp.float32) for _ in range(8))
        carry = (jnp.int32(-1),) + neg_row

        def loop_body(t, c):
            wait_in()

            @pl.when(t + 1 < ntot)
            def _():
                start_in(t + 1, (t + 1) % 2)

            return process(t, t % 2, c)

        carry = lax.fori_loop(0, ntot, loop_body, carry)
        close_seg(carry[0], carry[1:])

        pltpu.sync_copy(loc_v, zvert_hbm.at[pl.ds(v0 * 128, VPW * 128)])

    return seg_kernel(z_flat, vid_pad, bnd)


def _gather_add_sc(ze_flat, E, zv2, vid_pad, bnd):
    mesh = plsc.VectorSubcoreMesh(core_axis_name="c", subcore_axis_name="s")

    @functools.partial(
        pl.kernel,
        out_type=jax.ShapeDtypeStruct((E * 128,), jnp.float32),
        mesh=mesh,
        scratch_types=[
            pltpu.VMEM((48,), jnp.int32),
            pltpu.VMEM((TI + 16,), jnp.int32),
            pltpu.VMEM((TI + 16,), jnp.int32),
            pltpu.VMEM((T,), jnp.int32),
            pltpu.VMEM((T,), jnp.int32),
            pltpu.VMEM((T, 128), jnp.float32),
            pltpu.VMEM((T, 128), jnp.float32),
            pltpu.VMEM((T * 128,), jnp.float32),
            pltpu.VMEM((T * 128,), jnp.float32),
            pltpu.VMEM((T * 128,), jnp.float32),
            pltpu.VMEM((T * 128,), jnp.float32),
            pltpu.SemaphoreType.DMA,
            pltpu.SemaphoreType.DMA,
            pltpu.SemaphoreType.DMA,
            pltpu.SemaphoreType.DMA,
            pltpu.SemaphoreType.DMA,
            pltpu.SemaphoreType.DMA,
            pltpu.SemaphoreType.DMA,
            pltpu.SemaphoreType.DMA,
        ],
    )
    def ga_kernel(ze_hbm, zv2_hbm, vidp_hbm, bnd_hbm, out_hbm,
                  bnd_v, ids0, ids1, gidx0, gidx1, grows0, grows1,
                  rows0, rows1, ob0, ob1,
                  si0, si1, sg0, sg1, sr0, sr1, so0, so1):
        ids_b = (ids0, ids1)
        gidx_b = (gidx0, gidx1)
        grows_b = (grows0, grows1)
        rows_b = (rows0, rows1)
        out_b = (ob0, ob1)
        sem_i = (si0, si1)
        sem_g = (sg0, sg1)
        sem_r = (sr0, sr1)
        sem_o = (so0, so1)

        w = lax.axis_index("s") * 2 + lax.axis_index("c")
        v0 = w * VPW
        pltpu.sync_copy(bnd_hbm, bnd_v.at[pl.ds(0, 40)])
        e0 = _sload(bnd_v, w)
        e1 = _sload(bnd_v, w + 1)
        cnt = e1 - e0
        ntot = jnp.where(cnt >= T, (cnt + T - 1) // T, 0)

        def s_of(t):
            return jnp.minimum(e0 + t * T, e1 - T)

        def start_ids(t, b):
            s = s_of(t)
            s8 = (s // 8) * 8
            pltpu.async_copy(vidp_hbm.at[pl.ds(s8, TI)],
                             ids_b[b].at[pl.ds(0, TI)], sem_i[b])

        def wait_ids(b):
            pltpu.make_async_copy(vidp_hbm.at[pl.ds(0, TI)],
                                  ids_b[b].at[pl.ds(0, TI)], sem_i[b]).wait()

        def build_gidx(t, b):
            s = s_of(t)
            off = s - (s // 8) * 8

            def chunk(k, c):
                gidx_b[b][pl.ds(k * 16, 16)] = ids_b[b][pl.ds(off + k * 16, 16)]
                return c
            lax.fori_loop(0, T // 16, chunk, 0)

        def start_gather(t, b):
            s = s_of(t)
            pltpu.async_copy(zv2_hbm.at[gidx_b[b]], grows_b[b], sem_g[b])
            pltpu.async_copy(ze_hbm.at[pl.ds(s * 128, T * 128)], rows_b[b], sem_r[b])

        def wait_gather(b):
            pltpu.make_async_copy(zv2_hbm.at[gidx_b[b]], grows_b[b], sem_g[b]).wait()
            pltpu.make_async_copy(ze_hbm.at[pl.ds(0, T * 128)], rows_b[b], sem_r[b]).wait()

        def wait_out(b):
            pltpu.make_async_copy(out_b[b], out_hbm.at[pl.ds(0, T * 128)], sem_o[b]).wait()

        def process_add(b, r_lo, r_hi):
            def row_body(r, c):
                ro = r * 128
                for g in range(8):
                    out_b[b][pl.ds(ro + g * 16, 16)] = (
                        rows_b[b][pl.ds(ro + g * 16, 16)]
                        + grows_b[b][r, pl.ds(g * 16, 16)]
                    )
                return c
            lax.fori_loop(r_lo, r_hi, row_body, 0)

        def turn(t, b):
            wait_gather(b)

            @pl.when(t + 1 < ntot)
            def _():
                wait_ids(1 - b)
                build_gidx(t + 1, 1 - b)
                start_gather(t + 1, 1 - b)

            @pl.when(t + 2 < ntot)
            def _():
                start_ids(t + 2, b)

            @pl.when(t >= 2)
            def _():
                wait_out(b)

            process_add(b, 0, T)
            s = s_of(t)
            pltpu.async_copy(out_b[b], out_hbm.at[pl.ds(s * 128, T * 128)], sem_o[b])

        @pl.when(ntot > 0)
        def _():
            start_ids(0, 0)
            wait_ids(0)
            build_gidx(0, 0)
            start_gather(0, 0)

        @pl.when(ntot > 1)
        def _():
            start_ids(1, 1)

        def loop_body(t, c):
            lax.cond(
                t % 2 == 0,
                lambda: turn(t, 0),
                lambda: turn(t, 1),
            )
            return c

        lax.fori_loop(0, ntot, loop_body, 0)

        @pl.when(ntot >= 1)
        def _():
            wait_out(0)

        @pl.when(ntot >= 2)
        def _():
            wait_out(1)

        @pl.when((cnt > 0) & (cnt < T))
        def _():
            s = jnp.minimum(e0, E - T)
            s8 = (s // 8) * 8
            skip = e0 - s
            pltpu.sync_copy(vidp_hbm.at[pl.ds(s8, TI)], ids0.at[pl.ds(0, TI)])
            off = s - s8

            def chunk(k, c):
                gidx0[pl.ds(k * 16, 16)] = ids0[pl.ds(off + k * 16, 16)]
                return c
            lax.fori_loop(0, T // 16, chunk, 0)
            pltpu.async_copy(zv2_hbm.at[gidx0], grows0, sg0).wait()
            pltpu.sync_copy(ze_hbm.at[pl.ds(s * 128, T * 128)], rows0)
            process_add(0, skip, skip + cnt)

            def row_out(j, c):
                pltpu.sync_copy(
                    ob0.at[pl.ds(j * 128, 128)],
                    out_hbm.at[pl.ds((s + j) * 128, 128)],
                )
                return c
            lax.fori_loop(skip, skip + cnt, row_out, 0)

    return ga_kernel(ze_flat, zv2, vid_pad, bnd)


def kernel(x, vertex_id, W1, b1, We, Wv):
    E, C = x.shape
    vid = vertex_id.astype(jnp.int32)

    bnd = jnp.searchsorted(
        vid, (jnp.arange(33, dtype=jnp.int32) * VPW).astype(jnp.int32), side="left"
    ).astype(jnp.int32)
    bnd = jnp.concatenate([bnd, jnp.full((7,), E, jnp.int32)])
    vid_pad = jnp.concatenate([vid, jnp.zeros((88,), jnp.int32)])

    z, z_edge = _dual_matmul(x, W1, b1, We)
    zvert_flat = _seg_max_sc(z.reshape(-1), E, vid_pad, bnd)
    zvert = zvert_flat.reshape(VPAD, 128)
    zv2 = _small_matmul(zvert, Wv)
    out_flat = _gather_add_sc(z_edge.reshape(-1), E, zv2, vid_pad, bnd)
    return out_flat.reshape(E, 128)


# speedup vs baseline: 2.3527x; 1.0826x over previous
import functools
import jax
import jax.numpy as jnp
from jax import lax
from jax.experimental import pallas as pl
from jax.experimental.pallas import tpu as pltpu
from jax.experimental.pallas import tpu_sc as plsc

NW = 32
VPW = 313
VPAD = NW * VPW
T = 128
NEG = float("-inf")


def _sload(ref, i):
    return ref[pl.ds(i, 16)][0]


def _mmb_body(x_ref, wt_ref, b_ref, o_ref):
    o_ref[...] = (
        jnp.dot(x_ref[...], wt_ref[...], preferred_element_type=jnp.float32)
        + b_ref[...]
    )


def _matmul_bias(x, W, b, block_rows=2560):
    E, C = x.shape
    M = W.shape[0]
    return pl.pallas_call(
        _mmb_body,
        grid=(E // block_rows,),
        in_specs=[
            pl.BlockSpec((block_rows, C), lambda i: (i, 0)),
            pl.BlockSpec((C, M), lambda i: (0, 0)),
            pl.BlockSpec((1, M), lambda i: (0, 0)),
        ],
        out_specs=pl.BlockSpec((block_rows, M), lambda i: (i, 0)),
        out_shape=jax.ShapeDtypeStruct((E, M), jnp.float32),
    )(x, W.T, b.reshape(1, -1))


def _mm_body(a_ref, bt_ref, o_ref):
    o_ref[...] = jnp.dot(a_ref[...], bt_ref[...], preferred_element_type=jnp.float32)


def _small_matmul(a, Wv, block_rows=2504):
    V, M = a.shape
    O = Wv.shape[0]
    return pl.pallas_call(
        _mm_body,
        grid=(V // block_rows,),
        in_specs=[
            pl.BlockSpec((block_rows, M), lambda i: (i, 0)),
            pl.BlockSpec((M, O), lambda i: (0, 0)),
        ],
        out_specs=pl.BlockSpec((block_rows, O), lambda i: (i, 0)),
        out_shape=jax.ShapeDtypeStruct((V, O), jnp.float32),
    )(a, Wv.T)


TI = T + 8
IDS_STRIDE = 160


def _seg_max_sc(z_flat, E, vid_pad, bnd):
    mesh = plsc.VectorSubcoreMesh(core_axis_name="c", subcore_axis_name="s")

    @functools.partial(
        pl.kernel,
        out_type=jax.ShapeDtypeStruct((VPAD * 128,), jnp.float32),
        mesh=mesh,
        scratch_types=[
            pltpu.VMEM((48,), jnp.int32),
            pltpu.VMEM((2 * IDS_STRIDE,), jnp.int32),
            pltpu.VMEM((2 * T * 128,), jnp.float32),
            pltpu.VMEM((VPW * 128,), jnp.float32),
            pltpu.SemaphoreType.DMA,
            pltpu.SemaphoreType.DMA,
        ],
    )
    def seg_kernel(z_hbm, vidp_hbm, bnd_hbm, zvert_hbm,
                   bnd_v, ids_v, rows_v, loc_v, si, sr):
        w = lax.axis_index("s") * 2 + lax.axis_index("c")
        v0 = w * VPW
        pltpu.sync_copy(bnd_hbm, bnd_v.at[pl.ds(0, 40)])
        e0 = _sload(bnd_v, w)
        e1 = _sload(bnd_v, w + 1)
        cnt = e1 - e0
        ntot = (cnt + T - 1) // T

        def init_body(i, c):
            loc_v[pl.ds(i * 16, 16)] = jnp.full((16,), NEG, jnp.float32)
            return c
        lax.fori_loop(0, VPW * 8, init_body, 0)

        def s_of(t):
            return jnp.minimum(e0 + t * T, E - T)

        def start_in(t, slot):
            s = s_of(t)
            s8 = (s // 8) * 8
            pltpu.async_copy(
                vidp_hbm.at[pl.ds(s8, TI)],
                ids_v.at[pl.ds(slot * IDS_STRIDE, TI)], si)
            pltpu.async_copy(
                z_hbm.at[pl.ds(s * 128, T * 128)],
                rows_v.at[pl.ds(slot * T * 128, T * 128)], sr)

        def wait_in():
            pltpu.make_async_copy(
                vidp_hbm.at[pl.ds(0, TI)], ids_v.at[pl.ds(0, TI)], si).wait()
            pltpu.make_async_copy(
                z_hbm.at[pl.ds(0, T * 128)], rows_v.at[pl.ds(0, T * 128)], sr).wait()

        def close_seg(prev_vid, acc):
            @pl.when(prev_vid >= 0)
            def _():
                po = (prev_vid - v0) * 128
                for g in range(8):
                    loc_v[pl.ds(po + g * 16, 16)] = acc[g]

        def process(t, slot, carry):
            s = s_of(t)
            off = slot * IDS_STRIDE + s - (s // 8) * 8
            base_r = slot * (T * 128)
            skip = (e0 + t * T) - s
            j_hi = skip + jnp.minimum(T, cnt - t * T)

            def edge_body(j, c):
                vj, prev_vid = c[0], c[1]
                acc = c[2:]
                vnext = _sload(ids_v, off + j + 1)
                changed = vj != prev_vid

                @pl.when(changed)
                def _():
                    close_seg(prev_vid, acc)

                new_acc = []
                for g in range(8):
                    row = rows_v[pl.ds(base_r + j * 128 + g * 16, 16)]
                    new_acc.append(jnp.where(changed, row, jnp.maximum(acc[g], row)))
                return (vnext, jnp.where(changed, vj, prev_vid)) + tuple(new_acc)

            cin = (_sload(ids_v, off + skip), carry[0]) + carry[1:]
            cout = lax.fori_loop(skip, j_hi, edge_body, cin)
            return cout[1:]

        @pl.when(ntot > 0)
        def _():
            start_in(0, 0)

        neg_row = tuple(jnp.full((16,), NEG, jnp.float32) for _ in range(8))
        carry = (jnp.int32(-1),) + neg_row

        def loop_body(t, c):
            wait_in()

            @pl.when(t + 1 < ntot)
            def _():
                start_in(t + 1, (t + 1) % 2)

            return process(t, t % 2, c)

        carry = lax.fori_loop(0, ntot, loop_body, carry)
        close_seg(carry[0], carry[1:])

        pltpu.sync_copy(loc_v, zvert_hbm.at[pl.ds(v0 * 128, VPW * 128)])

    return seg_kernel(z_flat, vid_pad, bnd)


def _gather_add_sc(ze_flat, E, zv2, vid_pad, bnd):
    mesh = plsc.VectorSubcoreMesh(core_axis_name="c", subcore_axis_name="s")

    @functools.partial(
        pl.kernel,
        out_type=jax.ShapeDtypeStruct((E * 128,), jnp.float32),
        mesh=mesh,
        scratch_types=[
            pltpu.VMEM((48,), jnp.int32),
            pltpu.VMEM((TI + 16,), jnp.int32),
            pltpu.VMEM((TI + 16,), jnp.int32),
            pltpu.VMEM((T,), jnp.int32),
            pltpu.VMEM((T,), jnp.int32),
            pltpu.VMEM((T, 128), jnp.float32),
            pltpu.VMEM((T, 128), jnp.float32),
            pltpu.VMEM((T * 128,), jnp.float32),
            pltpu.VMEM((T * 128,), jnp.float32),
            pltpu.VMEM((T * 128,), jnp.float32),
            pltpu.VMEM((T * 128,), jnp.float32),
            pltpu.SemaphoreType.DMA,
            pltpu.SemaphoreType.DMA,
            pltpu.SemaphoreType.DMA,
            pltpu.SemaphoreType.DMA,
            pltpu.SemaphoreType.DMA,
            pltpu.SemaphoreType.DMA,
            pltpu.SemaphoreType.DMA,
            pltpu.SemaphoreType.DMA,
        ],
    )
    def ga_kernel(ze_hbm, zv2_hbm, vidp_hbm, bnd_hbm, out_hbm,
                  bnd_v, ids0, ids1, gidx0, gidx1, grows0, grows1,
                  rows0, rows1, ob0, ob1,
                  si0, si1, sg0, sg1, sr0, sr1, so0, so1):
        ids_b = (ids0, ids1)
        gidx_b = (gidx0, gidx1)
        grows_b = (grows0, grows1)
        rows_b = (rows0, rows1)
        out_b = (ob0, ob1)
        sem_i = (si0, si1)
        sem_g = (sg0, sg1)
        sem_r = (sr0, sr1)
        sem_o = (so0, so1)

        w = lax.axis_index("s") * 2 + lax.axis_index("c")
        v0 = w * VPW
        pltpu.sync_copy(bnd_hbm, bnd_v.at[pl.ds(0, 40)])
        e0 = _sload(bnd_v, w)
        e1 = _sload(bnd_v, w + 1)
        cnt = e1 - e0
        ntot = jnp.where(cnt >= T, (cnt + T - 1) // T, 0)

        def s_of(t):
            return jnp.minimum(e0 + t * T, e1 - T)

        def start_ids(t, b):
            s = s_of(t)
            s8 = (s // 8) * 8
            pltpu.async_copy(vidp_hbm.at[pl.ds(s8, TI)],
                             ids_b[b].at[pl.ds(0, TI)], sem_i[b])

        def wait_ids(b):
            pltpu.make_async_copy(vidp_hbm.at[pl.ds(0, TI)],
                                  ids_b[b].at[pl.ds(0, TI)], sem_i[b]).wait()

        def build_gidx(t, b):
            s = s_of(t)
            off = s - (s // 8) * 8

            def chunk(k, c):
                gidx_b[b][pl.ds(k * 16, 16)] = ids_b[b][pl.ds(off + k * 16, 16)]
                return c
            lax.fori_loop(0, T // 16, chunk, 0)

        def start_gather(t, b):
            s = s_of(t)
            pltpu.async_copy(zv2_hbm.at[gidx_b[b]], grows_b[b], sem_g[b])
            pltpu.async_copy(ze_hbm.at[pl.ds(s * 128, T * 128)], rows_b[b], sem_r[b])

        def wait_gather(b):
            pltpu.make_async_copy(zv2_hbm.at[gidx_b[b]], grows_b[b], sem_g[b]).wait()
            pltpu.make_async_copy(ze_hbm.at[pl.ds(0, T * 128)], rows_b[b], sem_r[b]).wait()

        def wait_out(b):
            pltpu.make_async_copy(out_b[b], out_hbm.at[pl.ds(0, T * 128)], sem_o[b]).wait()

        def process_add(b, r_lo, r_hi):
            def row_body(r, c):
                for u in range(2):
                    ro = (r * 2 + u) * 128
                    for g in range(8):
                        out_b[b][pl.ds(ro + g * 16, 16)] = (
                            rows_b[b][pl.ds(ro + g * 16, 16)]
                            + grows_b[b][r * 2 + u, pl.ds(g * 16, 16)]
                        )
                return c
            lax.fori_loop(r_lo, r_hi, row_body, 0)

        def process_add_tail(b, j_lo, j_hi):
            def row_body(r, c):
                ro = r * 128
                for g in range(8):
                    out_b[b][pl.ds(ro + g * 16, 16)] = (
                        rows_b[b][pl.ds(ro + g * 16, 16)]
                        + grows_b[b][r, pl.ds(g * 16, 16)]
                    )
                return c
            lax.fori_loop(j_lo, j_hi, row_body, 0)

        def turn(t, b):
            wait_gather(b)

            @pl.when(t + 1 < ntot)
            def _():
                wait_ids(1 - b)
                build_gidx(t + 1, 1 - b)
                start_gather(t + 1, 1 - b)

            @pl.when(t + 2 < ntot)
            def _():
                start_ids(t + 2, b)

            @pl.when(t >= 2)
            def _():
                wait_out(b)

            process_add(b, 0, T // 2)
            s = s_of(t)
            pltpu.async_copy(out_b[b], out_hbm.at[pl.ds(s * 128, T * 128)], sem_o[b])

        @pl.when(ntot > 0)
        def _():
            start_ids(0, 0)
            wait_ids(0)
            build_gidx(0, 0)
            start_gather(0, 0)

        @pl.when(ntot > 1)
        def _():
            start_ids(1, 1)

        def loop_body(t, c):
            lax.cond(
                t % 2 == 0,
                lambda: turn(t, 0),
                lambda: turn(t, 1),
            )
            return c

        lax.fori_loop(0, ntot, loop_body, 0)

        @pl.when(ntot >= 1)
        def _():
            wait_out(0)

        @pl.when(ntot >= 2)
        def _():
            wait_out(1)

        @pl.when((cnt > 0) & (cnt < T))
        def _():
            s = jnp.minimum(e0, E - T)
            s8 = (s // 8) * 8
            skip = e0 - s
            pltpu.sync_copy(vidp_hbm.at[pl.ds(s8, TI)], ids0.at[pl.ds(0, TI)])
            off = s - s8

            def chunk(k, c):
                gidx0[pl.ds(k * 16, 16)] = ids0[pl.ds(off + k * 16, 16)]
                return c
            lax.fori_loop(0, T // 16, chunk, 0)
            pltpu.async_copy(zv2_hbm.at[gidx0], grows0, sg0).wait()
            pltpu.sync_copy(ze_hbm.at[pl.ds(s * 128, T * 128)], rows0)
            process_add_tail(0, skip, skip + cnt)

            def row_out(j, c):
                pltpu.sync_copy(
                    ob0.at[pl.ds(j * 128, 128)],
                    out_hbm.at[pl.ds((s + j) * 128, 128)],
                )
                return c
            lax.fori_loop(skip, skip + cnt, row_out, 0)

    return ga_kernel(ze_flat, zv2, vid_pad, bnd)


def kernel(x, vertex_id, W1, b1, We, Wv):
    E, C = x.shape
    vid = vertex_id.astype(jnp.int32)

    bnd = jnp.searchsorted(
        vid, (jnp.arange(33, dtype=jnp.int32) * VPW).astype(jnp.int32), side="left"
    ).astype(jnp.int32)
    bnd = jnp.concatenate([bnd, jnp.full((7,), E, jnp.int32)])
    vid_pad = jnp.concatenate([vid, jnp.zeros((88,), jnp.int32)])

    z = _matmul_bias(x, W1, b1)
    zvert_flat = _seg_max_sc(z.reshape(-1), E, vid_pad, bnd)
    z_edge = _matmul_bias(x, We, jnp.zeros((128,), jnp.float32))
    zvert = zvert_flat.reshape(VPAD, 128)
    zv2 = _small_matmul(zvert, Wv)
    out_flat = _gather_add_sc(z_edge.reshape(-1), E, zv2, vid_pad, bnd)
    return out_flat.reshape(E, 128)


# speedup vs baseline: 2.8546x; 1.2133x over previous
import functools
import jax
import jax.numpy as jnp
from jax import lax
from jax.experimental import pallas as pl
from jax.experimental.pallas import tpu as pltpu
from jax.experimental.pallas import tpu_sc as plsc

NW = 32
VPW = 313
VPAD = NW * VPW
T = 128
NEG = float("-inf")


def _sload(ref, i):
    return ref[pl.ds(i, 16)][0]


def _mmb_body(x_ref, wt_ref, b_ref, o_ref):
    o_ref[...] = (
        jnp.dot(x_ref[...], wt_ref[...], preferred_element_type=jnp.float32)
        + b_ref[...]
    )


def _matmul_bias(x, W, b, block_rows=2560):
    E, C = x.shape
    M = W.shape[0]
    return pl.pallas_call(
        _mmb_body,
        grid=(E // block_rows,),
        in_specs=[
            pl.BlockSpec((block_rows, C), lambda i: (i, 0)),
            pl.BlockSpec((C, M), lambda i: (0, 0)),
            pl.BlockSpec((1, M), lambda i: (0, 0)),
        ],
        out_specs=pl.BlockSpec((block_rows, M), lambda i: (i, 0)),
        out_shape=jax.ShapeDtypeStruct((E, M), jnp.float32),
    )(x, W.T, b.reshape(1, -1))


def _mm_body(a_ref, bt_ref, o_ref):
    o_ref[...] = jnp.dot(a_ref[...], bt_ref[...], preferred_element_type=jnp.float32)


def _small_matmul(a, Wv, block_rows=2504):
    V, M = a.shape
    O = Wv.shape[0]
    return pl.pallas_call(
        _mm_body,
        grid=(V // block_rows,),
        in_specs=[
            pl.BlockSpec((block_rows, M), lambda i: (i, 0)),
            pl.BlockSpec((M, O), lambda i: (0, 0)),
        ],
        out_specs=pl.BlockSpec((block_rows, O), lambda i: (i, 0)),
        out_shape=jax.ShapeDtypeStruct((V, O), jnp.float32),
    )(a, Wv.T)


TI = T + 8
IDS_STRIDE = 160


def _seg_max_sc(z_flat, E, vid_pad, bnd):
    mesh = plsc.VectorSubcoreMesh(core_axis_name="c", subcore_axis_name="s")

    @functools.partial(
        pl.kernel,
        out_type=jax.ShapeDtypeStruct((VPAD * 128,), jnp.float32),
        mesh=mesh,
        scratch_types=[
            pltpu.VMEM((48,), jnp.int32),
            pltpu.VMEM((2 * IDS_STRIDE,), jnp.int32),
            pltpu.VMEM((2 * T * 128,), jnp.float32),
            pltpu.VMEM((VPW * 128,), jnp.float32),
            pltpu.SemaphoreType.DMA,
            pltpu.SemaphoreType.DMA,
        ],
    )
    def seg_kernel(z_hbm, vidp_hbm, bnd_hbm, zvert_hbm,
                   bnd_v, ids_v, rows_v, loc_v, si, sr):
        w = lax.axis_index("s") * 2 + lax.axis_index("c")
        v0 = w * VPW
        pltpu.sync_copy(bnd_hbm, bnd_v.at[pl.ds(0, 40)])
        e0 = _sload(bnd_v, w)
        e1 = _sload(bnd_v, w + 1)
        cnt = e1 - e0
        ntot = (cnt + T - 1) // T

        def init_body(i, c):
            loc_v[pl.ds(i * 16, 16)] = jnp.full((16,), NEG, jnp.float32)
            return c
        lax.fori_loop(0, VPW * 8, init_body, 0)

        def s_of(t):
            return jnp.minimum(e0 + t * T, E - T)

        def start_in(t, slot):
            s = s_of(t)
            s8 = (s // 8) * 8
            pltpu.async_copy(
                vidp_hbm.at[pl.ds(s8, TI)],
                ids_v.at[pl.ds(slot * IDS_STRIDE, TI)], si)
            pltpu.async_copy(
                z_hbm.at[pl.ds(s * 128, T * 128)],
                rows_v.at[pl.ds(slot * T * 128, T * 128)], sr)

        def wait_in():
            pltpu.make_async_copy(
                vidp_hbm.at[pl.ds(0, TI)], ids_v.at[pl.ds(0, TI)], si).wait()
            pltpu.make_async_copy(
                z_hbm.at[pl.ds(0, T * 128)], rows_v.at[pl.ds(0, T * 128)], sr).wait()

        def close_seg(prev_vid, acc):
            @pl.when(prev_vid >= 0)
            def _():
                po = (prev_vid - v0) * 128
                for g in range(8):
                    loc_v[pl.ds(po + g * 16, 16)] = acc[g]

        def process(t, slot, carry):
            s = s_of(t)
            off = slot * IDS_STRIDE + s - (s // 8) * 8
            base_r = slot * (T * 128)
            skip = (e0 + t * T) - s
            j_hi = skip + jnp.minimum(T, cnt - t * T)

            def edge_body(j, c):
                vj, prev_vid = c[0], c[1]
                acc = c[2:]
                vnext = _sload(ids_v, off + j + 1)
                changed = vj != prev_vid

                @pl.when(changed)
                def _():
                    close_seg(prev_vid, acc)

                new_acc = []
                for g in range(8):
                    row = rows_v[pl.ds(base_r + j * 128 + g * 16, 16)]
                    new_acc.append(jnp.where(changed, row, jnp.maximum(acc[g], row)))
                return (vnext, jnp.where(changed, vj, prev_vid)) + tuple(new_acc)

            cin = (_sload(ids_v, off + skip), carry[0]) + carry[1:]
            cout = lax.fori_loop(skip, j_hi, edge_body, cin)
            return cout[1:]

        @pl.when(ntot > 0)
        def _():
            start_in(0, 0)

        neg_row = tuple(jnp.full((16,), NEG, jnp.float32) for _ in range(8))
        carry = (jnp.int32(-1),) + neg_row

        def loop_body(t, c):
            wait_in()

            @pl.when(t + 1 < ntot)
            def _():
                start_in(t + 1, (t + 1) % 2)

            return process(t, t % 2, c)

        carry = lax.fori_loop(0, ntot, loop_body, carry)
        close_seg(carry[0], carry[1:])

        pltpu.sync_copy(loc_v, zvert_hbm.at[pl.ds(v0 * 128, VPW * 128)])

    return seg_kernel(z_flat, vid_pad, bnd)


def _gather_add_sc(ze_flat, E, zv2_flat, vid_pad, bnd):
    mesh = plsc.VectorSubcoreMesh(core_axis_name="c", subcore_axis_name="s")

    @functools.partial(
        pl.kernel,
        out_type=jax.ShapeDtypeStruct((E * 128,), jnp.float32),
        mesh=mesh,
        scratch_types=[
            pltpu.VMEM((48,), jnp.int32),
            pltpu.VMEM((TI + 16,), jnp.int32),
            pltpu.VMEM((TI + 16,), jnp.int32),
            pltpu.VMEM((T * 128,), jnp.float32),
            pltpu.VMEM((T * 128,), jnp.float32),
            pltpu.VMEM((T * 128,), jnp.float32),
            pltpu.VMEM((T * 128,), jnp.float32),
            pltpu.VMEM((VPW * 128,), jnp.float32),
            pltpu.SemaphoreType.DMA,
            pltpu.SemaphoreType.DMA,
            pltpu.SemaphoreType.DMA,
            pltpu.SemaphoreType.DMA,
            pltpu.SemaphoreType.DMA,
            pltpu.SemaphoreType.DMA,
        ],
    )
    def ga_kernel(ze_hbm, zv2_hbm, vidp_hbm, bnd_hbm, out_hbm,
                  bnd_v, ids0, ids1, rows0, rows1, ob0, ob1, loc_v,
                  si0, si1, sr0, sr1, so0, so1):
        ids_b = (ids0, ids1)
        rows_b = (rows0, rows1)
        out_b = (ob0, ob1)
        sem_i = (si0, si1)
        sem_r = (sr0, sr1)
        sem_o = (so0, so1)

        w = lax.axis_index("s") * 2 + lax.axis_index("c")
        v0 = w * VPW
        pltpu.sync_copy(bnd_hbm, bnd_v.at[pl.ds(0, 40)])
        e0 = _sload(bnd_v, w)
        e1 = _sload(bnd_v, w + 1)
        cnt = e1 - e0
        ntot = jnp.where(cnt >= T, (cnt + T - 1) // T, 0)

        pltpu.sync_copy(zv2_hbm.at[pl.ds(v0 * 128, VPW * 128)], loc_v)

        def s_of(t):
            return jnp.minimum(e0 + t * T, e1 - T)

        def start_in(t, b):
            s = s_of(t)
            s8 = (s // 8) * 8
            pltpu.async_copy(vidp_hbm.at[pl.ds(s8, TI)],
                             ids_b[b].at[pl.ds(0, TI)], sem_i[b])
            pltpu.async_copy(ze_hbm.at[pl.ds(s * 128, T * 128)],
                             rows_b[b], sem_r[b])

        def wait_in(b):
            pltpu.make_async_copy(vidp_hbm.at[pl.ds(0, TI)],
                                  ids_b[b].at[pl.ds(0, TI)], sem_i[b]).wait()
            pltpu.make_async_copy(ze_hbm.at[pl.ds(0, T * 128)],
                                  rows_b[b], sem_r[b]).wait()

        def wait_out(b):
            pltpu.make_async_copy(out_b[b], out_hbm.at[pl.ds(0, T * 128)],
                                  sem_o[b]).wait()

        def add_edge(b, j, po):
            ro = j * 128
            for g in range(8):
                out_b[b][pl.ds(ro + g * 16, 16)] = (
                    rows_b[b][pl.ds(ro + g * 16, 16)]
                    + loc_v[pl.ds(po + g * 16, 16)]
                )

        def slow_edges(b, off, j_lo, j_hi):
            def edge_body(j, c):
                vj = _sload(ids_b[b], off + j)
                add_edge(b, j, (vj - v0) * 128)
                return c
            lax.fori_loop(j_lo, j_hi, edge_body, 0)

        def process_tile(b, off):
            def group(k, c):
                v16 = ids_b[b][pl.ds(off + k * 16, 16)]
                va = v16[0]
                vb = v16[15]
                po = (va - v0) * 128

                @pl.when(va == vb)
                def _():
                    lr = [loc_v[pl.ds(po + g * 16, 16)] for g in range(8)]
                    for u in range(16):
                        ro = (k * 16 + u) * 128
                        for g in range(8):
                            out_b[b][pl.ds(ro + g * 16, 16)] = (
                                rows_b[b][pl.ds(ro + g * 16, 16)] + lr[g]
                            )

                @pl.when(va != vb)
                def _():
                    slow_edges(b, off, k * 16, k * 16 + 16)

                return c
            lax.fori_loop(0, T // 16, group, 0)

        def turn(t, b):
            wait_in(b)

            @pl.when(t + 1 < ntot)
            def _():
                start_in(t + 1, 1 - b)

            @pl.when(t >= 2)
            def _():
                wait_out(b)

            s = s_of(t)
            off = s - (s // 8) * 8
            process_tile(b, off)
            pltpu.async_copy(out_b[b], out_hbm.at[pl.ds(s * 128, T * 128)], sem_o[b])

        @pl.when(ntot > 0)
        def _():
            start_in(0, 0)

        def loop_body(t, c):
            lax.cond(
                t % 2 == 0,
                lambda: turn(t, 0),
                lambda: turn(t, 1),
            )
            return c

        lax.fori_loop(0, ntot, loop_body, 0)

        @pl.when(ntot >= 1)
        def _():
            wait_out(0)

        @pl.when(ntot >= 2)
        def _():
            wait_out(1)

        @pl.when((cnt > 0) & (cnt < T))
        def _():
            s = jnp.minimum(e0, E - T)
            s8 = (s // 8) * 8
            off = s - s8
            skip = e0 - s
            pltpu.sync_copy(vidp_hbm.at[pl.ds(s8, TI)], ids0.at[pl.ds(0, TI)])
            pltpu.sync_copy(ze_hbm.at[pl.ds(s * 128, T * 128)], rows0)
            slow_edges(0, off, skip, skip + cnt)

            def row_out(j, c):
                pltpu.sync_copy(
                    ob0.at[pl.ds(j * 128, 128)],
                    out_hbm.at[pl.ds((s + j) * 128, 128)],
                )
                return c
            lax.fori_loop(skip, skip + cnt, row_out, 0)

    return ga_kernel(ze_flat, zv2_flat, vid_pad, bnd)


def kernel(x, vertex_id, W1, b1, We, Wv):
    E, C = x.shape
    vid = vertex_id.astype(jnp.int32)

    bnd = jnp.searchsorted(
        vid, (jnp.arange(33, dtype=jnp.int32) * VPW).astype(jnp.int32), side="left"
    ).astype(jnp.int32)
    bnd = jnp.concatenate([bnd, jnp.full((7,), E, jnp.int32)])
    vid_pad = jnp.concatenate([vid, jnp.zeros((88,), jnp.int32)])

    z = _matmul_bias(x, W1, b1)
    zvert_flat = _seg_max_sc(z.reshape(-1), E, vid_pad, bnd)
    z_edge = _matmul_bias(x, We, jnp.zeros((128,), jnp.float32))
    zvert = zvert_flat.reshape(VPAD, 128)
    zv2 = _small_matmul(zvert, Wv)
    out_flat = _gather_add_sc(z_edge.reshape(-1), E, zv2.reshape(-1), vid_pad, bnd)
    return out_flat.reshape(E, 128)
